# Initial kernel scaffold; baseline (speedup 1.0000x reference)
import jax
import jax.numpy as jnp
from jax.experimental import pallas as pl


def kernel(x, edge_index, batch, params):
    raise NotImplementedError("write your pallas kernel here")


# speedup vs baseline: 5.1114x; 5.1114x over previous
import functools

import jax
import jax.numpy as jnp
from jax import lax
from jax.experimental import pallas as pl
from jax.experimental.pallas import tpu as pltpu
from jax.experimental.pallas import tpu_sc as plsc

N = 10000
D = 128
B = 64
N_PAD = 10240
NC = 2
NS = 16
NW = NC * NS
CH = 64
ZCH = 128
ROWS_PER_TILE = N_PAD // NS

R = 512
NB = N_PAD // R


def _sc_scatter_call(h, src2d, dst3d, zeros_blk):
    ept = src2d.shape[1]
    nchunk = ept // CH
    mesh = plsc.VectorSubcoreMesh(core_axis_name="c", subcore_axis_name="s")

    def body(h_hbm, src_hbm, dst_hbm, zeros_hbm, out_hbm,
             src_v, dst_v, rows_a, agg_sh, sem_a):
        c = lax.axis_index("c")
        s = lax.axis_index("s")
        w = s * NC + c
        pltpu.sync_copy(src_hbm.at[w], src_v)
        pltpu.sync_copy(dst_hbm.at[w], dst_v)
        base = s * ROWS_PER_TILE
        for r in range(ROWS_PER_TILE // ZCH):
            pltpu.sync_copy(zeros_hbm, agg_sh.at[pl.ds(base + r * ZCH, ZCH)])
        plsc.subcore_barrier()

        def step(j, carry):
            pltpu.async_copy(
                h_hbm.at[src_v.at[pl.ds(j * CH, CH)]], rows_a, sem_a).wait()
            pltpu.sync_copy(rows_a, agg_sh.at[dst_v.at[j]], add=True)
            return carry

        lax.fori_loop(0, nchunk, step, 0)
        plsc.subcore_barrier()
        pltpu.sync_copy(agg_sh.at[pl.ds(base, ROWS_PER_TILE)],
                        out_hbm.at[pl.ds(c * N_PAD + base, ROWS_PER_TILE)])

    kfn = pl.kernel(
        body,
        mesh=mesh,
        out_type=jax.ShapeDtypeStruct((NC * N_PAD, D), jnp.float32),
        scratch_types=[
            pltpu.VMEM((ept,), jnp.int32),
            pltpu.VMEM((nchunk, CH), jnp.int32),
            pltpu.VMEM((CH, D), jnp.float32),
            pltpu.VMEM_SHARED((N_PAD, D), jnp.float32),
            pltpu.SemaphoreType.DMA,
        ],
    )
    return kfn(h, src2d, dst3d, zeros_blk).reshape(NC, N_PAD, D)


def _mlp1_body(h_ref, a_ref, w_ref, b_ref, z_ref, mom_ref):
    i = pl.program_id(0)
    zin = h_ref[...] + a_ref[0] + a_ref[1]
    z = jnp.dot(zin, w_ref[...], preferred_element_type=jnp.float32) + b_ref[...]
    z_ref[...] = z
    rows = lax.broadcasted_iota(jnp.int32, (R, 1), 0) + i * R
    zm = jnp.where(rows < N, z, 0.0)
    mom = jnp.concatenate(
        [jnp.sum(zm, axis=0, keepdims=True),
         jnp.sum(zm * zm, axis=0, keepdims=True)], axis=0)

    @pl.when(i == 0)
    def _():
        mom_ref[...] = mom

    @pl.when(i > 0)
    def _():
        mom_ref[...] += mom


def _mlp1_call(h, agg, w1, b1):
    c = w1.shape[1]
    return pl.pallas_call(
        _mlp1_body,
        grid=(NB,),
        in_specs=[
            pl.BlockSpec((R, D), lambda i: (i, 0)),
            pl.BlockSpec((NC, R, D), lambda i: (0, i, 0)),
            pl.BlockSpec((D, c), lambda i: (0, 0)),
            pl.BlockSpec((1, c), lambda i: (0, 0)),
        ],
        out_specs=[
            pl.BlockSpec((R, c), lambda i: (i, 0)),
            pl.BlockSpec((2, c), lambda i: (0, 0)),
        ],
        out_shape=[
            jax.ShapeDtypeStruct((N_PAD, c), jnp.float32),
            jax.ShapeDtypeStruct((2, c), jnp.float32),
        ],
    )(h, agg, w1, b1.reshape(1, c))


def _mlp2_body(z_ref, mom_ref, g_ref, be_ref, w_ref, b_ref, h2_ref, mom2_ref):
    i = pl.program_id(0)
    mom = mom_ref[...]
    mean = mom[0:1] / N
    var = mom[1:2] / N - mean * mean
    rstd = lax.rsqrt(var + 1e-5)
    zn = (z_ref[...] - mean) * (rstd * g_ref[...]) + be_ref[...]
    zn = jnp.maximum(zn, 0.0)
    h2 = jnp.dot(zn, w_ref[...], preferred_element_type=jnp.float32) + b_ref[...]
    h2_ref[...] = h2
    rows = lax.broadcasted_iota(jnp.int32, (R, 1), 0) + i * R
    hm = jnp.where(rows < N, h2, 0.0)
    mom2 = jnp.concatenate(
        [jnp.sum(hm, axis=0, keepdims=True),
         jnp.sum(hm * hm, axis=0, keepdims=True)], axis=0)

    @pl.when(i == 0)
    def _():
        mom2_ref[...] = mom2

    @pl.when(i > 0)
    def _():
        mom2_ref[...] += mom2


def _mlp2_call(z, mom, g1, be1, w2, b2):
    c = z.shape[1]
    d2 = w2.shape[1]
    return pl.pallas_call(
        _mlp2_body,
        grid=(NB,),
        in_specs=[
            pl.BlockSpec((R, c), lambda i: (i, 0)),
            pl.BlockSpec((2, c), lambda i: (0, 0)),
            pl.BlockSpec((1, c), lambda i: (0, 0)),
            pl.BlockSpec((1, c), lambda i: (0, 0)),
            pl.BlockSpec((c, d2), lambda i: (0, 0)),
            pl.BlockSpec((1, d2), lambda i: (0, 0)),
        ],
        out_specs=[
            pl.BlockSpec((R, d2), lambda i: (i, 0)),
            pl.BlockSpec((2, d2), lambda i: (0, 0)),
        ],
        out_shape=[
            jax.ShapeDtypeStruct((N_PAD, d2), jnp.float32),
            jax.ShapeDtypeStruct((2, d2), jnp.float32),
        ],
    )(z, mom, g1.reshape(1, c), be1.reshape(1, c), w2, b2.reshape(1, d2))


def _bn_body(h2_ref, mom_ref, g_ref, be_ref, out_ref, *, relu):
    mom = mom_ref[...]
    mean = mom[0:1] / N
    var = mom[1:2] / N - mean * mean
    rstd = lax.rsqrt(var + 1e-5)
    h = (h2_ref[...] - mean) * (rstd * g_ref[...]) + be_ref[...]
    if relu:
        h = jnp.maximum(h, 0.0)
    out_ref[...] = h


def _bn_call(h2, mom, g2, be2, relu):
    return pl.pallas_call(
        functools.partial(_bn_body, relu=relu),
        grid=(NB,),
        in_specs=[
            pl.BlockSpec((R, D), lambda i: (i, 0)),
            pl.BlockSpec((2, D), lambda i: (0, 0)),
            pl.BlockSpec((1, D), lambda i: (0, 0)),
            pl.BlockSpec((1, D), lambda i: (0, 0)),
        ],
        out_specs=pl.BlockSpec((R, D), lambda i: (i, 0)),
        out_shape=jax.ShapeDtypeStruct((N_PAD, D), jnp.float32),
    )(h2, mom, g2.reshape(1, D), be2.reshape(1, D))


def _readout_body(h_ref, bt_ref, cw_ref, cb_ref, out_ref, sums, cnts):
    i = pl.program_id(0)

    @pl.when(i == 0)
    def _():
        sums[...] = jnp.zeros_like(sums)
        cnts[...] = jnp.zeros_like(cnts)

    bt = bt_ref[0, 0]
    oh = (lax.broadcasted_iota(jnp.int32, (B, R), 0)
          == bt[None, :]).astype(jnp.float32)
    sums[...] += jnp.dot(oh, h_ref[...], preferred_element_type=jnp.float32)
    cnts[...] += jnp.broadcast_to(jnp.sum(oh, axis=1, keepdims=True), cnts.shape)

    @pl.when(i == NB - 1)
    def _():
        ro = sums[...] / jnp.maximum(cnts[...][:, 0:1], 1.0)
        out_ref[...] = jnp.dot(
            ro, cw_ref[...], preferred_element_type=jnp.float32) + cb_ref[...]


def _readout_call(h, batch3d, cw, cb):
    out = cw.shape[1]
    return pl.pallas_call(
        _readout_body,
        grid=(NB,),
        in_specs=[
            pl.BlockSpec((R, D), lambda i: (i, 0)),
            pl.BlockSpec((1, 1, R), lambda i: (i, 0, 0)),
            pl.BlockSpec((D, out), lambda i: (0, 0)),
            pl.BlockSpec((1, out), lambda i: (0, 0)),
        ],
        out_specs=pl.BlockSpec((B, out), lambda i: (0, 0)),
        out_shape=jax.ShapeDtypeStruct((B, out), jnp.float32),
        scratch_shapes=[
            pltpu.VMEM((B, D), jnp.float32),
            pltpu.VMEM((B, 128), jnp.float32),
        ],
    )(h, batch3d, cw, cb.reshape(1, out))


def kernel(x, edge_index, batch, params):
    e = edge_index.shape[1]
    ept = -(-e // (NW * CH)) * CH
    e_pad = ept * NW
    npad = N_PAD - N

    src = edge_index[0]
    dst = edge_index[1]
    pad_idx = (jnp.arange(e_pad - e, dtype=jnp.int32) % npad) + N
    src2d = jnp.concatenate([src, pad_idx]).reshape(NW, ept)
    dst3d = jnp.concatenate([dst, pad_idx]).reshape(NW, ept // CH, CH)

    h = jnp.concatenate([x, jnp.zeros((npad, D), jnp.float32)], axis=0)
    batch3d = jnp.concatenate(
        [batch.astype(jnp.int32), jnp.full((npad,), B, jnp.int32)]
    ).reshape(NB, 1, R)
    zeros_blk = jnp.zeros((ZCH, D), jnp.float32)

    layers = params['layers']
    for li, p in enumerate(layers):
        agg = _sc_scatter_call(h, src2d, dst3d, zeros_blk)
        z1, mom1 = _mlp1_call(h, agg, p['W1'], p['b1'])
        h2, mom2 = _mlp2_call(z1, mom1, p['g1'], p['be1'], p['W2'], p['b2'])
        h = _bn_call(h2, mom2, p['g2'], p['be2'], relu=li != len(layers) - 1)

    return _readout_call(h, batch3d, params['cls_W'], params['cls_b'])


# speedup vs baseline: 6.2447x; 1.2217x over previous
import functools

import jax
import jax.numpy as jnp
from jax import lax
from jax.experimental import pallas as pl
from jax.experimental.pallas import tpu as pltpu
from jax.experimental.pallas import tpu_sc as plsc

N = 10000
D = 128
B = 64
N_PAD = 10240
NC = 2
NS = 16
NW = NC * NS
CH = 128
ZCH = 128
ROWS_PER_TILE = N_PAD // NS

R = 512
NB = N_PAD // R


def _sc_scatter_call(h, src2d, dst3d, zeros_blk):
    ept = src2d.shape[1]
    nchunk = ept // CH
    mesh = plsc.VectorSubcoreMesh(core_axis_name="c", subcore_axis_name="s")

    def body(h_hbm, src_hbm, dst_hbm, zeros_hbm, out_hbm,
             src_v, dst_v, rows_a, agg_sh, sem_a):
        c = lax.axis_index("c")
        s = lax.axis_index("s")
        w = s * NC + c
        pltpu.sync_copy(src_hbm.at[w], src_v)
        pltpu.sync_copy(dst_hbm.at[w], dst_v)
        base = s * ROWS_PER_TILE
        for r in range(ROWS_PER_TILE // ZCH):
            pltpu.sync_copy(zeros_hbm, agg_sh.at[pl.ds(base + r * ZCH, ZCH)])
        plsc.subcore_barrier()

        def step(j, carry):
            pltpu.async_copy(
                h_hbm.at[src_v.at[pl.ds(j * CH, CH)]], rows_a, sem_a).wait()
            pltpu.sync_copy(rows_a, agg_sh.at[dst_v.at[j]], add=True)
            return carry

        lax.fori_loop(0, nchunk, step, 0)
        plsc.subcore_barrier()
        pltpu.sync_copy(agg_sh.at[pl.ds(base, ROWS_PER_TILE)],
                        out_hbm.at[pl.ds(c * N_PAD + base, ROWS_PER_TILE)])

    kfn = pl.kernel(
        body,
        mesh=mesh,
        out_type=jax.ShapeDtypeStruct((NC * N_PAD, D), jnp.float32),
        scratch_types=[
            pltpu.VMEM((ept,), jnp.int32),
            pltpu.VMEM((nchunk, CH), jnp.int32),
            pltpu.VMEM((CH, D), jnp.float32),
            pltpu.VMEM_SHARED((N_PAD, D), jnp.float32),
            pltpu.SemaphoreType.DMA,
        ],
    )
    return kfn(h, src2d, dst3d, zeros_blk).reshape(NC, N_PAD, D)


def _mlp1_body(h_ref, a_ref, w_ref, b_ref, z_ref, mom_ref):
    i = pl.program_id(0)
    zin = h_ref[...] + a_ref[0] + a_ref[1]
    z = jnp.dot(zin, w_ref[...], preferred_element_type=jnp.float32) + b_ref[...]
    z_ref[...] = z
    rows = lax.broadcasted_iota(jnp.int32, (R, 1), 0) + i * R
    zm = jnp.where(rows < N, z, 0.0)
    mom = jnp.concatenate(
        [jnp.sum(zm, axis=0, keepdims=True),
         jnp.sum(zm * zm, axis=0, keepdims=True)], axis=0)

    @pl.when(i == 0)
    def _():
        mom_ref[...] = mom

    @pl.when(i > 0)
    def _():
        mom_ref[...] += mom


def _mlp1_call(h, agg, w1, b1):
    c = w1.shape[1]
    return pl.pallas_call(
        _mlp1_body,
        grid=(NB,),
        in_specs=[
            pl.BlockSpec((R, D), lambda i: (i, 0)),
            pl.BlockSpec((NC, R, D), lambda i: (0, i, 0)),
            pl.BlockSpec((D, c), lambda i: (0, 0)),
            pl.BlockSpec((1, c), lambda i: (0, 0)),
        ],
        out_specs=[
            pl.BlockSpec((R, c), lambda i: (i, 0)),
            pl.BlockSpec((2, c), lambda i: (0, 0)),
        ],
        out_shape=[
            jax.ShapeDtypeStruct((N_PAD, c), jnp.float32),
            jax.ShapeDtypeStruct((2, c), jnp.float32),
        ],
    )(h, agg, w1, b1.reshape(1, c))


def _mlp2_body(z_ref, mom_ref, g_ref, be_ref, w_ref, b_ref, h2_ref, mom2_ref):
    i = pl.program_id(0)
    mom = mom_ref[...]
    mean = mom[0:1] / N
    var = mom[1:2] / N - mean * mean
    rstd = lax.rsqrt(var + 1e-5)
    zn = (z_ref[...] - mean) * (rstd * g_ref[...]) + be_ref[...]
    zn = jnp.maximum(zn, 0.0)
    h2 = jnp.dot(zn, w_ref[...], preferred_element_type=jnp.float32) + b_ref[...]
    h2_ref[...] = h2
    rows = lax.broadcasted_iota(jnp.int32, (R, 1), 0) + i * R
    hm = jnp.where(rows < N, h2, 0.0)
    mom2 = jnp.concatenate(
        [jnp.sum(hm, axis=0, keepdims=True),
         jnp.sum(hm * hm, axis=0, keepdims=True)], axis=0)

    @pl.when(i == 0)
    def _():
        mom2_ref[...] = mom2

    @pl.when(i > 0)
    def _():
        mom2_ref[...] += mom2


def _mlp2_call(z, mom, g1, be1, w2, b2):
    c = z.shape[1]
    d2 = w2.shape[1]
    return pl.pallas_call(
        _mlp2_body,
        grid=(NB,),
        in_specs=[
            pl.BlockSpec((R, c), lambda i: (i, 0)),
            pl.BlockSpec((2, c), lambda i: (0, 0)),
            pl.BlockSpec((1, c), lambda i: (0, 0)),
            pl.BlockSpec((1, c), lambda i: (0, 0)),
            pl.BlockSpec((c, d2), lambda i: (0, 0)),
            pl.BlockSpec((1, d2), lambda i: (0, 0)),
        ],
        out_specs=[
            pl.BlockSpec((R, d2), lambda i: (i, 0)),
            pl.BlockSpec((2, d2), lambda i: (0, 0)),
        ],
        out_shape=[
            jax.ShapeDtypeStruct((N_PAD, d2), jnp.float32),
            jax.ShapeDtypeStruct((2, d2), jnp.float32),
        ],
    )(z, mom, g1.reshape(1, c), be1.reshape(1, c), w2, b2.reshape(1, d2))


def _bn_body(h2_ref, mom_ref, g_ref, be_ref, out_ref, *, relu):
    mom = mom_ref[...]
    mean = mom[0:1] / N
    var = mom[1:2] / N - mean * mean
    rstd = lax.rsqrt(var + 1e-5)
    h = (h2_ref[...] - mean) * (rstd * g_ref[...]) + be_ref[...]
    if relu:
        h = jnp.maximum(h, 0.0)
    out_ref[...] = h


def _bn_call(h2, mom, g2, be2, relu):
    return pl.pallas_call(
        functools.partial(_bn_body, relu=relu),
        grid=(NB,),
        in_specs=[
            pl.BlockSpec((R, D), lambda i: (i, 0)),
            pl.BlockSpec((2, D), lambda i: (0, 0)),
            pl.BlockSpec((1, D), lambda i: (0, 0)),
            pl.BlockSpec((1, D), lambda i: (0, 0)),
        ],
        out_specs=pl.BlockSpec((R, D), lambda i: (i, 0)),
        out_shape=jax.ShapeDtypeStruct((N_PAD, D), jnp.float32),
    )(h2, mom, g2.reshape(1, D), be2.reshape(1, D))


def _readout_body(h_ref, bt_ref, cw_ref, cb_ref, out_ref, sums, cnts):
    i = pl.program_id(0)

    @pl.when(i == 0)
    def _():
        sums[...] = jnp.zeros_like(sums)
        cnts[...] = jnp.zeros_like(cnts)

    bt = bt_ref[0, 0]
    oh = (lax.broadcasted_iota(jnp.int32, (B, R), 0)
          == bt[None, :]).astype(jnp.float32)
    sums[...] += jnp.dot(oh, h_ref[...], preferred_element_type=jnp.float32)
    cnts[...] += jnp.broadcast_to(jnp.sum(oh, axis=1, keepdims=True), cnts.shape)

    @pl.when(i == NB - 1)
    def _():
        ro = sums[...] / jnp.maximum(cnts[...][:, 0:1], 1.0)
        out_ref[...] = jnp.dot(
            ro, cw_ref[...], preferred_element_type=jnp.float32) + cb_ref[...]


def _readout_call(h, batch3d, cw, cb):
    out = cw.shape[1]
    return pl.pallas_call(
        _readout_body,
        grid=(NB,),
        in_specs=[
            pl.BlockSpec((R, D), lambda i: (i, 0)),
            pl.BlockSpec((1, 1, R), lambda i: (i, 0, 0)),
            pl.BlockSpec((D, out), lambda i: (0, 0)),
            pl.BlockSpec((1, out), lambda i: (0, 0)),
        ],
        out_specs=pl.BlockSpec((B, out), lambda i: (0, 0)),
        out_shape=jax.ShapeDtypeStruct((B, out), jnp.float32),
        scratch_shapes=[
            pltpu.VMEM((B, D), jnp.float32),
            pltpu.VMEM((B, 128), jnp.float32),
        ],
    )(h, batch3d, cw, cb.reshape(1, out))


def kernel(x, edge_index, batch, params):
    e = edge_index.shape[1]
    ept = -(-e // (NW * CH)) * CH
    e_pad = ept * NW
    npad = N_PAD - N

    src = edge_index[0]
    dst = edge_index[1]
    pad_idx = (jnp.arange(e_pad - e, dtype=jnp.int32) % npad) + N
    src2d = jnp.concatenate([src, pad_idx]).reshape(NW, ept)
    dst3d = jnp.concatenate([dst, pad_idx]).reshape(NW, ept // CH, CH)

    h = jnp.concatenate([x, jnp.zeros((npad, D), jnp.float32)], axis=0)
    batch3d = jnp.concatenate(
        [batch.astype(jnp.int32), jnp.full((npad,), B, jnp.int32)]
    ).reshape(NB, 1, R)
    zeros_blk = jnp.zeros((ZCH, D), jnp.float32)

    layers = params['layers']
    for li, p in enumerate(layers):
        agg = _sc_scatter_call(h, src2d, dst3d, zeros_blk)
        z1, mom1 = _mlp1_call(h, agg, p['W1'], p['b1'])
        h2, mom2 = _mlp2_call(z1, mom1, p['g1'], p['be1'], p['W2'], p['b2'])
        h = _bn_call(h2, mom2, p['g2'], p['be2'], relu=li != len(layers) - 1)

    return _readout_call(h, batch3d, params['cls_W'], params['cls_b'])


# speedup vs baseline: 6.8143x; 1.0912x over previous
import functools

import jax
import jax.numpy as jnp
from jax import lax
from jax.experimental import pallas as pl
from jax.experimental.pallas import tpu as pltpu
from jax.experimental.pallas import tpu_sc as plsc

N = 10000
D = 128
B = 64
N_AGG = 10240
NC = 2
NS = 16
NW = NC * NS
CH = 128
ZCH = 128
ROWS_PER_TILE = N_AGG // NS

R = 1000
NB = N // R


def _sc_scatter_call(h, src2d, dst3d, zeros_blk):
    ept = src2d.shape[1]
    nchunk = ept // CH
    mesh = plsc.VectorSubcoreMesh(core_axis_name="c", subcore_axis_name="s")

    def body(h_hbm, src_hbm, dst_hbm, zeros_hbm, out_hbm,
             src_v, dst_v, rows_a, agg_sh, sem_a):
        c = lax.axis_index("c")
        s = lax.axis_index("s")
        w = s * NC + c
        pltpu.sync_copy(src_hbm.at[w], src_v)
        pltpu.sync_copy(dst_hbm.at[w], dst_v)
        base = s * ROWS_PER_TILE
        for r in range(ROWS_PER_TILE // ZCH):
            pltpu.sync_copy(zeros_hbm, agg_sh.at[pl.ds(base + r * ZCH, ZCH)])
        plsc.subcore_barrier()

        def step(j, carry):
            pltpu.async_copy(
                h_hbm.at[src_v.at[pl.ds(j * CH, CH)]], rows_a, sem_a).wait()
            pltpu.sync_copy(rows_a, agg_sh.at[dst_v.at[j]], add=True)
            return carry

        lax.fori_loop(0, nchunk, step, 0)
        plsc.subcore_barrier()
        pltpu.sync_copy(agg_sh.at[pl.ds(base, ROWS_PER_TILE)],
                        out_hbm.at[pl.ds(c * N_AGG + base, ROWS_PER_TILE)])

    kfn = pl.kernel(
        body,
        mesh=mesh,
        out_type=jax.ShapeDtypeStruct((NC * N_AGG, D), jnp.float32),
        scratch_types=[
            pltpu.VMEM((ept,), jnp.int32),
            pltpu.VMEM((nchunk, CH), jnp.int32),
            pltpu.VMEM((CH, D), jnp.float32),
            pltpu.VMEM_SHARED((N_AGG, D), jnp.float32),
            pltpu.SemaphoreType.DMA,
        ],
    )
    return kfn(h, src2d, dst3d, zeros_blk).reshape(NC, N_AGG, D)


def _mlp1_body(h_ref, a_ref, w_ref, b_ref, z_ref, mom_ref):
    i = pl.program_id(0)
    zin = h_ref[...] + a_ref[0] + a_ref[1]
    z = jnp.dot(zin, w_ref[...], preferred_element_type=jnp.float32) + b_ref[...]
    z_ref[...] = z
    mom = jnp.concatenate(
        [jnp.sum(z, axis=0, keepdims=True),
         jnp.sum(z * z, axis=0, keepdims=True)], axis=0)

    @pl.when(i == 0)
    def _():
        mom_ref[...] = mom

    @pl.when(i > 0)
    def _():
        mom_ref[...] += mom


def _mlp1_call(h, agg, w1, b1):
    c = w1.shape[1]
    return pl.pallas_call(
        _mlp1_body,
        grid=(NB,),
        in_specs=[
            pl.BlockSpec((R, D), lambda i: (i, 0)),
            pl.BlockSpec((NC, R, D), lambda i: (0, i, 0)),
            pl.BlockSpec((D, c), lambda i: (0, 0)),
            pl.BlockSpec((1, c), lambda i: (0, 0)),
        ],
        out_specs=[
            pl.BlockSpec((R, c), lambda i: (i, 0)),
            pl.BlockSpec((2, c), lambda i: (0, 0)),
        ],
        out_shape=[
            jax.ShapeDtypeStruct((N, c), jnp.float32),
            jax.ShapeDtypeStruct((2, c), jnp.float32),
        ],
    )(h, agg, w1, b1.reshape(1, c))


def _mlp2_body(z_ref, mom_ref, g_ref, be_ref, w_ref, b_ref, h2_ref, mom2_ref):
    i = pl.program_id(0)
    mom = mom_ref[...]
    mean = mom[0:1] / N
    var = mom[1:2] / N - mean * mean
    rstd = lax.rsqrt(var + 1e-5)
    zn = (z_ref[...] - mean) * (rstd * g_ref[...]) + be_ref[...]
    zn = jnp.maximum(zn, 0.0)
    h2 = jnp.dot(zn, w_ref[...], preferred_element_type=jnp.float32) + b_ref[...]
    h2_ref[...] = h2
    mom2 = jnp.concatenate(
        [jnp.sum(h2, axis=0, keepdims=True),
         jnp.sum(h2 * h2, axis=0, keepdims=True)], axis=0)

    @pl.when(i == 0)
    def _():
        mom2_ref[...] = mom2

    @pl.when(i > 0)
    def _():
        mom2_ref[...] += mom2


def _mlp2_call(z, mom, g1, be1, w2, b2):
    c = z.shape[1]
    d2 = w2.shape[1]
    return pl.pallas_call(
        _mlp2_body,
        grid=(NB,),
        in_specs=[
            pl.BlockSpec((R, c), lambda i: (i, 0)),
            pl.BlockSpec((2, c), lambda i: (0, 0)),
            pl.BlockSpec((1, c), lambda i: (0, 0)),
            pl.BlockSpec((1, c), lambda i: (0, 0)),
            pl.BlockSpec((c, d2), lambda i: (0, 0)),
            pl.BlockSpec((1, d2), lambda i: (0, 0)),
        ],
        out_specs=[
            pl.BlockSpec((R, d2), lambda i: (i, 0)),
            pl.BlockSpec((2, d2), lambda i: (0, 0)),
        ],
        out_shape=[
            jax.ShapeDtypeStruct((N, d2), jnp.float32),
            jax.ShapeDtypeStruct((2, d2), jnp.float32),
        ],
    )(z, mom, g1.reshape(1, c), be1.reshape(1, c), w2, b2.reshape(1, d2))


def _bn_body(h2_ref, mom_ref, g_ref, be_ref, out_ref):
    mom = mom_ref[...]
    mean = mom[0:1] / N
    var = mom[1:2] / N - mean * mean
    rstd = lax.rsqrt(var + 1e-5)
    h = (h2_ref[...] - mean) * (rstd * g_ref[...]) + be_ref[...]
    out_ref[...] = jnp.maximum(h, 0.0)


def _bn_call(h2, mom, g2, be2):
    return pl.pallas_call(
        _bn_body,
        grid=(NB,),
        in_specs=[
            pl.BlockSpec((R, D), lambda i: (i, 0)),
            pl.BlockSpec((2, D), lambda i: (0, 0)),
            pl.BlockSpec((1, D), lambda i: (0, 0)),
            pl.BlockSpec((1, D), lambda i: (0, 0)),
        ],
        out_specs=pl.BlockSpec((R, D), lambda i: (i, 0)),
        out_shape=jax.ShapeDtypeStruct((N, D), jnp.float32),
    )(h2, mom, g2.reshape(1, D), be2.reshape(1, D))


def _bn_readout_body(h2_ref, mom_ref, g_ref, be_ref, bt_ref, cw_ref, cb_ref,
                     out_ref, sums, cnts):
    i = pl.program_id(0)

    @pl.when(i == 0)
    def _():
        sums[...] = jnp.zeros_like(sums)
        cnts[...] = jnp.zeros_like(cnts)

    mom = mom_ref[...]
    mean = mom[0:1] / N
    var = mom[1:2] / N - mean * mean
    rstd = lax.rsqrt(var + 1e-5)
    h = (h2_ref[...] - mean) * (rstd * g_ref[...]) + be_ref[...]

    bt = bt_ref[0, 0]
    oh = (lax.broadcasted_iota(jnp.int32, (B, R), 0)
          == bt[None, :]).astype(jnp.float32)
    sums[...] += jnp.dot(oh, h, preferred_element_type=jnp.float32)
    cnts[...] += jnp.broadcast_to(jnp.sum(oh, axis=1, keepdims=True), cnts.shape)

    @pl.when(i == NB - 1)
    def _():
        ro = sums[...] / jnp.maximum(cnts[...][:, 0:1], 1.0)
        out_ref[...] = jnp.dot(
            ro, cw_ref[...], preferred_element_type=jnp.float32) + cb_ref[...]


def _bn_readout_call(h2, mom, g2, be2, batch3d, cw, cb):
    out = cw.shape[1]
    return pl.pallas_call(
        _bn_readout_body,
        grid=(NB,),
        in_specs=[
            pl.BlockSpec((R, D), lambda i: (i, 0)),
            pl.BlockSpec((2, D), lambda i: (0, 0)),
            pl.BlockSpec((1, D), lambda i: (0, 0)),
            pl.BlockSpec((1, D), lambda i: (0, 0)),
            pl.BlockSpec((1, 1, R), lambda i: (i, 0, 0)),
            pl.BlockSpec((D, out), lambda i: (0, 0)),
            pl.BlockSpec((1, out), lambda i: (0, 0)),
        ],
        out_specs=pl.BlockSpec((B, out), lambda i: (0, 0)),
        out_shape=jax.ShapeDtypeStruct((B, out), jnp.float32),
        scratch_shapes=[
            pltpu.VMEM((B, D), jnp.float32),
            pltpu.VMEM((B, 128), jnp.float32),
        ],
    )(h2, mom, g2.reshape(1, D), be2.reshape(1, D), batch3d, cw,
      cb.reshape(1, out))


def kernel(x, edge_index, batch, params):
    e = edge_index.shape[1]
    ept = -(-e // (NW * CH)) * CH
    e_pad = ept * NW
    npad = N_AGG - N

    src = edge_index[0]
    dst = edge_index[1]
    pad_idx = jnp.arange(e_pad - e, dtype=jnp.int32) % npad
    src2d = jnp.concatenate([src, pad_idx]).reshape(NW, ept)
    dst3d = jnp.concatenate([dst, pad_idx + N]).reshape(NW, ept // CH, CH)

    batch3d = batch.astype(jnp.int32).reshape(NB, 1, R)
    zeros_blk = jnp.zeros((ZCH, D), jnp.float32)

    h = x
    layers = params['layers']
    for li, p in enumerate(layers):
        agg = _sc_scatter_call(h, src2d, dst3d, zeros_blk)
        z1, mom1 = _mlp1_call(h, agg, p['W1'], p['b1'])
        h2, mom2 = _mlp2_call(z1, mom1, p['g1'], p['be1'], p['W2'], p['b2'])
        if li != len(layers) - 1:
            h = _bn_call(h2, mom2, p['g2'], p['be2'])
        else:
            return _bn_readout_call(h2, mom2, p['g2'], p['be2'], batch3d,
                                    params['cls_W'], params['cls_b'])


# speedup vs baseline: 7.8463x; 1.1514x over previous
import jax
import jax.numpy as jnp
from jax import lax
from jax.experimental import pallas as pl
from jax.experimental.pallas import tpu as pltpu
from jax.experimental.pallas import tpu_sc as plsc

N = 10000
D = 128
B = 64
N_AGG = 10240
NC = 2
NS = 16
NW = NC * NS
CH = 128
ROWS_PER_TILE = N_AGG // NS

R = 1000
NB = N // R


def _sc_scatter_call(h, src2d, dst3d):
    ept_half = src2d.shape[1]
    nchalf = ept_half // CH
    mesh = plsc.VectorSubcoreMesh(core_axis_name="c", subcore_axis_name="s")

    def body(h_hbm, src_hbm, dst_hbm, out_hbm,
             src_v, dst_v, rows_a, rows_b, agg_sh, sem_a):
        c = lax.axis_index("c")
        s = lax.axis_index("s")
        w = s * NC + c
        def zrow(r, carry):
            for jj in range(D // 16):
                rows_a[r, pl.ds(jj * 16, 16)] = jnp.zeros((16,), jnp.float32)
            return carry

        lax.fori_loop(0, CH, zrow, 0)
        base = s * ROWS_PER_TILE
        for r in range(ROWS_PER_TILE // CH):
            pltpu.sync_copy(rows_a, agg_sh.at[pl.ds(base + r * CH, CH)])
        plsc.subcore_barrier()

        def step(j, carry):
            ca = pltpu.async_copy(
                h_hbm.at[src_v.at[pl.ds((2 * j) * CH, CH)]], rows_a, sem_a)
            cb = pltpu.async_copy(
                h_hbm.at[src_v.at[pl.ds((2 * j + 1) * CH, CH)]], rows_b, sem_a)
            ca.wait()
            cb.wait()
            pltpu.sync_copy(rows_a, agg_sh.at[dst_v.at[2 * j]], add=True)
            pltpu.sync_copy(rows_b, agg_sh.at[dst_v.at[2 * j + 1]], add=True)
            return carry

        for k in range(2):
            pltpu.sync_copy(src_hbm.at[w * 2 + k], src_v)
            pltpu.sync_copy(dst_hbm.at[w * 2 + k], dst_v)
            lax.fori_loop(0, nchalf // 2, step, 0)
        plsc.subcore_barrier()
        pltpu.sync_copy(agg_sh.at[pl.ds(base, ROWS_PER_TILE)],
                        out_hbm.at[pl.ds(c * N_AGG + base, ROWS_PER_TILE)])

    kfn = pl.kernel(
        body,
        mesh=mesh,
        out_type=jax.ShapeDtypeStruct((NC * N_AGG, D), jnp.float32),
        scratch_types=[
            pltpu.VMEM((ept_half,), jnp.int32),
            pltpu.VMEM((nchalf, CH), jnp.int32),
            pltpu.VMEM((CH, D), jnp.float32),
            pltpu.VMEM((CH, D), jnp.float32),
            pltpu.VMEM_SHARED((N_AGG, D), jnp.float32),
            pltpu.SemaphoreType.DMA,
        ],
    )
    return kfn(h, src2d, dst3d).reshape(NC, N_AGG, D)


def _mlp1_body(h_ref, a_ref, w_ref, b_ref, z_ref, mom_ref):
    i = pl.program_id(0)
    zin = h_ref[...] + a_ref[0] + a_ref[1]
    z = jnp.dot(zin, w_ref[...], preferred_element_type=jnp.float32) + b_ref[...]
    z_ref[...] = z
    mom = jnp.concatenate(
        [jnp.sum(z, axis=0, keepdims=True),
         jnp.sum(z * z, axis=0, keepdims=True)], axis=0)

    @pl.when(i == 0)
    def _():
        mom_ref[...] = mom

    @pl.when(i > 0)
    def _():
        mom_ref[...] += mom


def _mlp1_call(h, agg, w1, b1):
    c = w1.shape[1]
    return pl.pallas_call(
        _mlp1_body,
        grid=(NB,),
        in_specs=[
            pl.BlockSpec((R, D), lambda i: (i, 0)),
            pl.BlockSpec((NC, R, D), lambda i: (0, i, 0)),
            pl.BlockSpec((D, c), lambda i: (0, 0)),
            pl.BlockSpec((1, c), lambda i: (0, 0)),
        ],
        out_specs=[
            pl.BlockSpec((R, c), lambda i: (i, 0)),
            pl.BlockSpec((2, c), lambda i: (0, 0)),
        ],
        out_shape=[
            jax.ShapeDtypeStruct((N, c), jnp.float32),
            jax.ShapeDtypeStruct((2, c), jnp.float32),
        ],
    )(h, agg, w1, b1.reshape(1, c))


def _mlp2_body(z_ref, mom_ref, g_ref, be_ref, w_ref, b_ref, h2_ref, mom2_ref):
    i = pl.program_id(0)
    mom = mom_ref[...]
    mean = mom[0:1] / N
    var = mom[1:2] / N - mean * mean
    rstd = lax.rsqrt(var + 1e-5)
    zn = (z_ref[...] - mean) * (rstd * g_ref[...]) + be_ref[...]
    zn = jnp.maximum(zn, 0.0)
    h2 = jnp.dot(zn, w_ref[...], preferred_element_type=jnp.float32) + b_ref[...]
    h2_ref[...] = h2
    mom2 = jnp.concatenate(
        [jnp.sum(h2, axis=0, keepdims=True),
         jnp.sum(h2 * h2, axis=0, keepdims=True)], axis=0)

    @pl.when(i == 0)
    def _():
        mom2_ref[...] = mom2

    @pl.when(i > 0)
    def _():
        mom2_ref[...] += mom2


def _mlp2_call(z, mom, g1, be1, w2, b2):
    c = z.shape[1]
    d2 = w2.shape[1]
    return pl.pallas_call(
        _mlp2_body,
        grid=(NB,),
        in_specs=[
            pl.BlockSpec((R, c), lambda i: (i, 0)),
            pl.BlockSpec((2, c), lambda i: (0, 0)),
            pl.BlockSpec((1, c), lambda i: (0, 0)),
            pl.BlockSpec((1, c), lambda i: (0, 0)),
            pl.BlockSpec((c, d2), lambda i: (0, 0)),
            pl.BlockSpec((1, d2), lambda i: (0, 0)),
        ],
        out_specs=[
            pl.BlockSpec((R, d2), lambda i: (i, 0)),
            pl.BlockSpec((2, d2), lambda i: (0, 0)),
        ],
        out_shape=[
            jax.ShapeDtypeStruct((N, d2), jnp.float32),
            jax.ShapeDtypeStruct((2, d2), jnp.float32),
        ],
    )(z, mom, g1.reshape(1, c), be1.reshape(1, c), w2, b2.reshape(1, d2))


def _bn_body(h2_ref, mom_ref, g_ref, be_ref, out_ref):
    mom = mom_ref[...]
    mean = mom[0:1] / N
    var = mom[1:2] / N - mean * mean
    rstd = lax.rsqrt(var + 1e-5)
    h = (h2_ref[...] - mean) * (rstd * g_ref[...]) + be_ref[...]
    out_ref[...] = jnp.maximum(h, 0.0)


def _bn_call(h2, mom, g2, be2):
    return pl.pallas_call(
        _bn_body,
        grid=(NB,),
        in_specs=[
            pl.BlockSpec((R, D), lambda i: (i, 0)),
            pl.BlockSpec((2, D), lambda i: (0, 0)),
            pl.BlockSpec((1, D), lambda i: (0, 0)),
            pl.BlockSpec((1, D), lambda i: (0, 0)),
        ],
        out_specs=pl.BlockSpec((R, D), lambda i: (i, 0)),
        out_shape=jax.ShapeDtypeStruct((N, D), jnp.float32),
    )(h2, mom, g2.reshape(1, D), be2.reshape(1, D))


def _bn_readout_body(h2_ref, mom_ref, g_ref, be_ref, bt_ref, cw_ref, cb_ref,
                     out_ref, sums, cnts):
    i = pl.program_id(0)

    @pl.when(i == 0)
    def _():
        sums[...] = jnp.zeros_like(sums)
        cnts[...] = jnp.zeros_like(cnts)

    mom = mom_ref[...]
    mean = mom[0:1] / N
    var = mom[1:2] / N - mean * mean
    rstd = lax.rsqrt(var + 1e-5)
    h = (h2_ref[...] - mean) * (rstd * g_ref[...]) + be_ref[...]

    bt = bt_ref[0, 0]
    oh = (lax.broadcasted_iota(jnp.int32, (B, R), 0)
          == bt[None, :]).astype(jnp.float32)
    sums[...] += jnp.dot(oh, h, preferred_element_type=jnp.float32)
    cnts[...] += jnp.broadcast_to(jnp.sum(oh, axis=1, keepdims=True), cnts.shape)

    @pl.when(i == NB - 1)
    def _():
        ro = sums[...] / jnp.maximum(cnts[...][:, 0:1], 1.0)
        out_ref[...] = jnp.dot(
            ro, cw_ref[...], preferred_element_type=jnp.float32) + cb_ref[...]


def _bn_readout_call(h2, mom, g2, be2, batch3d, cw, cb):
    out = cw.shape[1]
    return pl.pallas_call(
        _bn_readout_body,
        grid=(NB,),
        in_specs=[
            pl.BlockSpec((R, D), lambda i: (i, 0)),
            pl.BlockSpec((2, D), lambda i: (0, 0)),
            pl.BlockSpec((1, D), lambda i: (0, 0)),
            pl.BlockSpec((1, D), lambda i: (0, 0)),
            pl.BlockSpec((1, 1, R), lambda i: (i, 0, 0)),
            pl.BlockSpec((D, out), lambda i: (0, 0)),
            pl.BlockSpec((1, out), lambda i: (0, 0)),
        ],
        out_specs=pl.BlockSpec((B, out), lambda i: (0, 0)),
        out_shape=jax.ShapeDtypeStruct((B, out), jnp.float32),
        scratch_shapes=[
            pltpu.VMEM((B, D), jnp.float32),
            pltpu.VMEM((B, 128), jnp.float32),
        ],
    )(h2, mom, g2.reshape(1, D), be2.reshape(1, D), batch3d, cw,
      cb.reshape(1, out))


def kernel(x, edge_index, batch, params):
    e = edge_index.shape[1]
    ept = -(-e // (NW * 2 * CH)) * 2 * CH
    e_pad = ept * NW
    npad = N_AGG - N

    src = edge_index[0]
    dst = edge_index[1]
    pad_idx = jnp.arange(e_pad - e, dtype=jnp.int32) % npad
    src2d = jnp.concatenate([src, pad_idx]).reshape(NW * 2, ept // 2)
    dst3d = jnp.concatenate([dst, pad_idx + N]).reshape(
        NW * 2, ept // (2 * CH), CH)

    batch3d = batch.astype(jnp.int32).reshape(NB, 1, R)

    h = x
    layers = params['layers']
    for li, p in enumerate(layers):
        agg = _sc_scatter_call(h, src2d, dst3d)
        z1, mom1 = _mlp1_call(h, agg, p['W1'], p['b1'])
        h2, mom2 = _mlp2_call(z1, mom1, p['g1'], p['be1'], p['W2'], p['b2'])
        if li != len(layers) - 1:
            h = _bn_call(h2, mom2, p['g2'], p['be2'])
        else:
            return _bn_readout_call(h2, mom2, p['g2'], p['be2'], batch3d,
                                    params['cls_W'], params['cls_b'])


# speedup vs baseline: 7.9442x; 1.0125x over previous
import jax
import jax.numpy as jnp
from jax import lax
from jax.experimental import pallas as pl
from jax.experimental.pallas import tpu as pltpu
from jax.experimental.pallas import tpu_sc as plsc

N = 10000
D = 128
B = 64
N_AGG = 10240
NC = 2
NS = 16
NW = NC * NS
CH = 128
ROWS_PER_TILE = N_AGG // NS

R = 1000
NB = N // R


def _sc_scatter_call(h, src2d, dst3d):
    ept_half = src2d.shape[1]
    nchalf = ept_half // CH
    mesh = plsc.VectorSubcoreMesh(core_axis_name="c", subcore_axis_name="s")

    def body(h_hbm, src_hbm, dst_hbm, out_hbm,
             src_v, dst_v, rows_a, rows_b, agg_sh, sem_a, sem_s):
        c = lax.axis_index("c")
        s = lax.axis_index("s")
        w = s * NC + c
        def zrow(r, carry):
            for jj in range(D // 16):
                rows_a[r, pl.ds(jj * 16, 16)] = jnp.zeros((16,), jnp.float32)
            return carry

        lax.fori_loop(0, CH, zrow, 0)
        base = s * ROWS_PER_TILE
        for r in range(ROWS_PER_TILE // CH):
            pltpu.sync_copy(rows_a, agg_sh.at[pl.ds(base + r * CH, CH)])
        plsc.subcore_barrier()

        def step(j, carry):
            ca = pltpu.async_copy(
                h_hbm.at[src_v.at[pl.ds((2 * j) * CH, CH)]], rows_a, sem_a)
            cb = pltpu.async_copy(
                h_hbm.at[src_v.at[pl.ds((2 * j + 1) * CH, CH)]], rows_b, sem_a)
            ca.wait()
            cb.wait()
            sa = pltpu.async_copy(
                rows_a, agg_sh.at[dst_v.at[2 * j]], sem_s, add=True)
            sb = pltpu.async_copy(
                rows_b, agg_sh.at[dst_v.at[2 * j + 1]], sem_s, add=True)
            sa.wait()
            sb.wait()
            return carry

        for k in range(2):
            pltpu.sync_copy(src_hbm.at[w * 2 + k], src_v)
            pltpu.sync_copy(dst_hbm.at[w * 2 + k], dst_v)
            lax.fori_loop(0, nchalf // 2, step, 0)
        plsc.subcore_barrier()
        pltpu.sync_copy(agg_sh.at[pl.ds(base, ROWS_PER_TILE)],
                        out_hbm.at[pl.ds(c * N_AGG + base, ROWS_PER_TILE)])

    kfn = pl.kernel(
        body,
        mesh=mesh,
        out_type=jax.ShapeDtypeStruct((NC * N_AGG, D), jnp.float32),
        scratch_types=[
            pltpu.VMEM((ept_half,), jnp.int32),
            pltpu.VMEM((nchalf, CH), jnp.int32),
            pltpu.VMEM((CH, D), jnp.float32),
            pltpu.VMEM((CH, D), jnp.float32),
            pltpu.VMEM_SHARED((N_AGG, D), jnp.float32),
            pltpu.SemaphoreType.DMA,
            pltpu.SemaphoreType.DMA,
        ],
    )
    return kfn(h, src2d, dst3d).reshape(NC, N_AGG, D)


def _mlp1_body(h_ref, a_ref, w_ref, b_ref, z_ref, mom_ref):
    i = pl.program_id(0)
    zin = h_ref[...] + a_ref[0] + a_ref[1]
    z = jnp.dot(zin, w_ref[...], preferred_element_type=jnp.float32) + b_ref[...]
    z_ref[...] = z
    mom = jnp.concatenate(
        [jnp.sum(z, axis=0, keepdims=True),
         jnp.sum(z * z, axis=0, keepdims=True)], axis=0)

    @pl.when(i == 0)
    def _():
        mom_ref[...] = mom

    @pl.when(i > 0)
    def _():
        mom_ref[...] += mom


def _mlp1_call(h, agg, w1, b1):
    c = w1.shape[1]
    return pl.pallas_call(
        _mlp1_body,
        grid=(NB,),
        in_specs=[
            pl.BlockSpec((R, D), lambda i: (i, 0)),
            pl.BlockSpec((NC, R, D), lambda i: (0, i, 0)),
            pl.BlockSpec((D, c), lambda i: (0, 0)),
            pl.BlockSpec((1, c), lambda i: (0, 0)),
        ],
        out_specs=[
            pl.BlockSpec((R, c), lambda i: (i, 0)),
            pl.BlockSpec((2, c), lambda i: (0, 0)),
        ],
        out_shape=[
            jax.ShapeDtypeStruct((N, c), jnp.float32),
            jax.ShapeDtypeStruct((2, c), jnp.float32),
        ],
    )(h, agg, w1, b1.reshape(1, c))


def _mlp2_body(z_ref, mom_ref, g_ref, be_ref, w_ref, b_ref, h2_ref, mom2_ref):
    i = pl.program_id(0)
    mom = mom_ref[...]
    mean = mom[0:1] / N
    var = mom[1:2] / N - mean * mean
    rstd = lax.rsqrt(var + 1e-5)
    zn = (z_ref[...] - mean) * (rstd * g_ref[...]) + be_ref[...]
    zn = jnp.maximum(zn, 0.0)
    h2 = jnp.dot(zn, w_ref[...], preferred_element_type=jnp.float32) + b_ref[...]
    h2_ref[...] = h2
    mom2 = jnp.concatenate(
        [jnp.sum(h2, axis=0, keepdims=True),
         jnp.sum(h2 * h2, axis=0, keepdims=True)], axis=0)

    @pl.when(i == 0)
    def _():
        mom2_ref[...] = mom2

    @pl.when(i > 0)
    def _():
        mom2_ref[...] += mom2


def _mlp2_call(z, mom, g1, be1, w2, b2):
    c = z.shape[1]
    d2 = w2.shape[1]
    return pl.pallas_call(
        _mlp2_body,
        grid=(NB,),
        in_specs=[
            pl.BlockSpec((R, c), lambda i: (i, 0)),
            pl.BlockSpec((2, c), lambda i: (0, 0)),
            pl.BlockSpec((1, c), lambda i: (0, 0)),
            pl.BlockSpec((1, c), lambda i: (0, 0)),
            pl.BlockSpec((c, d2), lambda i: (0, 0)),
            pl.BlockSpec((1, d2), lambda i: (0, 0)),
        ],
        out_specs=[
            pl.BlockSpec((R, d2), lambda i: (i, 0)),
            pl.BlockSpec((2, d2), lambda i: (0, 0)),
        ],
        out_shape=[
            jax.ShapeDtypeStruct((N, d2), jnp.float32),
            jax.ShapeDtypeStruct((2, d2), jnp.float32),
        ],
    )(z, mom, g1.reshape(1, c), be1.reshape(1, c), w2, b2.reshape(1, d2))


def _bn_body(h2_ref, mom_ref, g_ref, be_ref, out_ref):
    mom = mom_ref[...]
    mean = mom[0:1] / N
    var = mom[1:2] / N - mean * mean
    rstd = lax.rsqrt(var + 1e-5)
    h = (h2_ref[...] - mean) * (rstd * g_ref[...]) + be_ref[...]
    out_ref[...] = jnp.maximum(h, 0.0)


def _bn_call(h2, mom, g2, be2):
    return pl.pallas_call(
        _bn_body,
        grid=(NB,),
        in_specs=[
            pl.BlockSpec((R, D), lambda i: (i, 0)),
            pl.BlockSpec((2, D), lambda i: (0, 0)),
            pl.BlockSpec((1, D), lambda i: (0, 0)),
            pl.BlockSpec((1, D), lambda i: (0, 0)),
        ],
        out_specs=pl.BlockSpec((R, D), lambda i: (i, 0)),
        out_shape=jax.ShapeDtypeStruct((N, D), jnp.float32),
    )(h2, mom, g2.reshape(1, D), be2.reshape(1, D))


def _bn_readout_body(h2_ref, mom_ref, g_ref, be_ref, bt_ref, cw_ref, cb_ref,
                     out_ref, sums, cnts):
    i = pl.program_id(0)

    @pl.when(i == 0)
    def _():
        sums[...] = jnp.zeros_like(sums)
        cnts[...] = jnp.zeros_like(cnts)

    mom = mom_ref[...]
    mean = mom[0:1] / N
    var = mom[1:2] / N - mean * mean
    rstd = lax.rsqrt(var + 1e-5)
    h = (h2_ref[...] - mean) * (rstd * g_ref[...]) + be_ref[...]

    bt = bt_ref[0, 0]
    oh = (lax.broadcasted_iota(jnp.int32, (B, R), 0)
          == bt[None, :]).astype(jnp.float32)
    sums[...] += jnp.dot(oh, h, preferred_element_type=jnp.float32)
    cnts[...] += jnp.broadcast_to(jnp.sum(oh, axis=1, keepdims=True), cnts.shape)

    @pl.when(i == NB - 1)
    def _():
        ro = sums[...] / jnp.maximum(cnts[...][:, 0:1], 1.0)
        out_ref[...] = jnp.dot(
            ro, cw_ref[...], preferred_element_type=jnp.float32) + cb_ref[...]


def _bn_readout_call(h2, mom, g2, be2, batch3d, cw, cb):
    out = cw.shape[1]
    return pl.pallas_call(
        _bn_readout_body,
        grid=(NB,),
        in_specs=[
            pl.BlockSpec((R, D), lambda i: (i, 0)),
            pl.BlockSpec((2, D), lambda i: (0, 0)),
            pl.BlockSpec((1, D), lambda i: (0, 0)),
            pl.BlockSpec((1, D), lambda i: (0, 0)),
            pl.BlockSpec((1, 1, R), lambda i: (i, 0, 0)),
            pl.BlockSpec((D, out), lambda i: (0, 0)),
            pl.BlockSpec((1, out), lambda i: (0, 0)),
        ],
        out_specs=pl.BlockSpec((B, out), lambda i: (0, 0)),
        out_shape=jax.ShapeDtypeStruct((B, out), jnp.float32),
        scratch_shapes=[
            pltpu.VMEM((B, D), jnp.float32),
            pltpu.VMEM((B, 128), jnp.float32),
        ],
    )(h2, mom, g2.reshape(1, D), be2.reshape(1, D), batch3d, cw,
      cb.reshape(1, out))


def kernel(x, edge_index, batch, params):
    e = edge_index.shape[1]
    ept = -(-e // (NW * 2 * CH)) * 2 * CH
    e_pad = ept * NW
    npad = N_AGG - N

    src = edge_index[0]
    dst = edge_index[1]
    pad_idx = jnp.arange(e_pad - e, dtype=jnp.int32) % npad
    src2d = jnp.concatenate([src, pad_idx]).reshape(NW * 2, ept // 2)
    dst3d = jnp.concatenate([dst, pad_idx + N]).reshape(
        NW * 2, ept // (2 * CH), CH)

    batch3d = batch.astype(jnp.int32).reshape(NB, 1, R)

    h = x
    layers = params['layers']
    for li, p in enumerate(layers):
        agg = _sc_scatter_call(h, src2d, dst3d)
        z1, mom1 = _mlp1_call(h, agg, p['W1'], p['b1'])
        h2, mom2 = _mlp2_call(z1, mom1, p['g1'], p['be1'], p['W2'], p['b2'])
        if li != len(layers) - 1:
            h = _bn_call(h2, mom2, p['g2'], p['be2'])
        else:
            return _bn_readout_call(h2, mom2, p['g2'], p['be2'], batch3d,
                                    params['cls_W'], params['cls_b'])


# speedup vs baseline: 8.1790x; 1.0296x over previous
import jax
import jax.numpy as jnp
from jax import lax
from jax.experimental import pallas as pl
from jax.experimental.pallas import tpu as pltpu
from jax.experimental.pallas import tpu_sc as plsc

N = 10000
D = 128
B = 64
N_AGG = 10240
NC = 2
NS = 16
NW = NC * NS
CH = 128
ROWS_PER_TILE = N_AGG // NS

R = 2000
NB = N // R


def _sc_scatter_call(h, src2d, dst3d):
    ept_half = src2d.shape[1]
    nchalf = ept_half // CH
    mesh = plsc.VectorSubcoreMesh(core_axis_name="c", subcore_axis_name="s")

    def body(h_hbm, src_hbm, dst_hbm, out_hbm,
             src_v, dst_v, rows_a, rows_b, agg_sh, sem_a, sem_s):
        c = lax.axis_index("c")
        s = lax.axis_index("s")
        w = s * NC + c
        def zrow(r, carry):
            for jj in range(D // 16):
                rows_a[r, pl.ds(jj * 16, 16)] = jnp.zeros((16,), jnp.float32)
            return carry

        lax.fori_loop(0, CH, zrow, 0)
        base = s * ROWS_PER_TILE
        for r in range(ROWS_PER_TILE // CH):
            pltpu.sync_copy(rows_a, agg_sh.at[pl.ds(base + r * CH, CH)])
        plsc.subcore_barrier()

        def step(j, carry):
            ca = pltpu.async_copy(
                h_hbm.at[src_v.at[pl.ds((2 * j) * CH, CH)]], rows_a, sem_a)
            cb = pltpu.async_copy(
                h_hbm.at[src_v.at[pl.ds((2 * j + 1) * CH, CH)]], rows_b, sem_a)
            ca.wait()
            cb.wait()
            sa = pltpu.async_copy(
                rows_a, agg_sh.at[dst_v.at[2 * j]], sem_s, add=True)
            sb = pltpu.async_copy(
                rows_b, agg_sh.at[dst_v.at[2 * j + 1]], sem_s, add=True)
            sa.wait()
            sb.wait()
            return carry

        for k in range(2):
            pltpu.sync_copy(src_hbm.at[w * 2 + k], src_v)
            pltpu.sync_copy(dst_hbm.at[w * 2 + k], dst_v)
            lax.fori_loop(0, nchalf // 2, step, 0)
        plsc.subcore_barrier()
        pltpu.sync_copy(agg_sh.at[pl.ds(base, ROWS_PER_TILE)],
                        out_hbm.at[pl.ds(c * N_AGG + base, ROWS_PER_TILE)])

    kfn = pl.kernel(
        body,
        mesh=mesh,
        out_type=jax.ShapeDtypeStruct((NC * N_AGG, D), jnp.float32),
        scratch_types=[
            pltpu.VMEM((ept_half,), jnp.int32),
            pltpu.VMEM((nchalf, CH), jnp.int32),
            pltpu.VMEM((CH, D), jnp.float32),
            pltpu.VMEM((CH, D), jnp.float32),
            pltpu.VMEM_SHARED((N_AGG, D), jnp.float32),
            pltpu.SemaphoreType.DMA,
            pltpu.SemaphoreType.DMA,
        ],
    )
    return kfn(h, src2d, dst3d).reshape(NC, N_AGG, D)


def _mlp1_body(h_ref, a_ref, w_ref, b_ref, z_ref, mom_ref):
    i = pl.program_id(0)
    zin = h_ref[...] + a_ref[0] + a_ref[1]
    z = jnp.dot(zin, w_ref[...], preferred_element_type=jnp.float32) + b_ref[...]
    z_ref[...] = z
    mom = jnp.concatenate(
        [jnp.sum(z, axis=0, keepdims=True),
         jnp.sum(z * z, axis=0, keepdims=True)], axis=0)

    @pl.when(i == 0)
    def _():
        mom_ref[...] = mom

    @pl.when(i > 0)
    def _():
        mom_ref[...] += mom


def _mlp1_call(h, agg, w1, b1):
    c = w1.shape[1]
    return pl.pallas_call(
        _mlp1_body,
        grid=(NB,),
        in_specs=[
            pl.BlockSpec((R, D), lambda i: (i, 0)),
            pl.BlockSpec((NC, R, D), lambda i: (0, i, 0)),
            pl.BlockSpec((D, c), lambda i: (0, 0)),
            pl.BlockSpec((1, c), lambda i: (0, 0)),
        ],
        out_specs=[
            pl.BlockSpec((R, c), lambda i: (i, 0)),
            pl.BlockSpec((2, c), lambda i: (0, 0)),
        ],
        out_shape=[
            jax.ShapeDtypeStruct((N, c), jnp.float32),
            jax.ShapeDtypeStruct((2, c), jnp.float32),
        ],
    )(h, agg, w1, b1.reshape(1, c))


def _mlp2_body(z_ref, mom_ref, g_ref, be_ref, w_ref, b_ref, h2_ref, mom2_ref):
    i = pl.program_id(0)
    mom = mom_ref[...]
    mean = mom[0:1] / N
    var = mom[1:2] / N - mean * mean
    rstd = lax.rsqrt(var + 1e-5)
    zn = (z_ref[...] - mean) * (rstd * g_ref[...]) + be_ref[...]
    zn = jnp.maximum(zn, 0.0)
    h2 = jnp.dot(zn, w_ref[...], preferred_element_type=jnp.float32) + b_ref[...]
    h2_ref[...] = h2
    mom2 = jnp.concatenate(
        [jnp.sum(h2, axis=0, keepdims=True),
         jnp.sum(h2 * h2, axis=0, keepdims=True)], axis=0)

    @pl.when(i == 0)
    def _():
        mom2_ref[...] = mom2

    @pl.when(i > 0)
    def _():
        mom2_ref[...] += mom2


def _mlp2_call(z, mom, g1, be1, w2, b2):
    c = z.shape[1]
    d2 = w2.shape[1]
    return pl.pallas_call(
        _mlp2_body,
        grid=(NB,),
        in_specs=[
            pl.BlockSpec((R, c), lambda i: (i, 0)),
            pl.BlockSpec((2, c), lambda i: (0, 0)),
            pl.BlockSpec((1, c), lambda i: (0, 0)),
            pl.BlockSpec((1, c), lambda i: (0, 0)),
            pl.BlockSpec((c, d2), lambda i: (0, 0)),
            pl.BlockSpec((1, d2), lambda i: (0, 0)),
        ],
        out_specs=[
            pl.BlockSpec((R, d2), lambda i: (i, 0)),
            pl.BlockSpec((2, d2), lambda i: (0, 0)),
        ],
        out_shape=[
            jax.ShapeDtypeStruct((N, d2), jnp.float32),
            jax.ShapeDtypeStruct((2, d2), jnp.float32),
        ],
    )(z, mom, g1.reshape(1, c), be1.reshape(1, c), w2, b2.reshape(1, d2))


def _bn_body(h2_ref, mom_ref, g_ref, be_ref, out_ref):
    mom = mom_ref[...]
    mean = mom[0:1] / N
    var = mom[1:2] / N - mean * mean
    rstd = lax.rsqrt(var + 1e-5)
    h = (h2_ref[...] - mean) * (rstd * g_ref[...]) + be_ref[...]
    out_ref[...] = jnp.maximum(h, 0.0)


def _bn_call(h2, mom, g2, be2):
    return pl.pallas_call(
        _bn_body,
        grid=(NB,),
        in_specs=[
            pl.BlockSpec((R, D), lambda i: (i, 0)),
            pl.BlockSpec((2, D), lambda i: (0, 0)),
            pl.BlockSpec((1, D), lambda i: (0, 0)),
            pl.BlockSpec((1, D), lambda i: (0, 0)),
        ],
        out_specs=pl.BlockSpec((R, D), lambda i: (i, 0)),
        out_shape=jax.ShapeDtypeStruct((N, D), jnp.float32),
    )(h2, mom, g2.reshape(1, D), be2.reshape(1, D))


def _bn_readout_body(h2_ref, mom_ref, g_ref, be_ref, bt_ref, cw_ref, cb_ref,
                     out_ref, sums, cnts):
    i = pl.program_id(0)

    @pl.when(i == 0)
    def _():
        sums[...] = jnp.zeros_like(sums)
        cnts[...] = jnp.zeros_like(cnts)

    mom = mom_ref[...]
    mean = mom[0:1] / N
    var = mom[1:2] / N - mean * mean
    rstd = lax.rsqrt(var + 1e-5)
    h = (h2_ref[...] - mean) * (rstd * g_ref[...]) + be_ref[...]

    bt = bt_ref[0, 0]
    oh = (lax.broadcasted_iota(jnp.int32, (B, R), 0)
          == bt[None, :]).astype(jnp.float32)
    sums[...] += jnp.dot(oh, h, preferred_element_type=jnp.float32)
    cnts[...] += jnp.broadcast_to(jnp.sum(oh, axis=1, keepdims=True), cnts.shape)

    @pl.when(i == NB - 1)
    def _():
        ro = sums[...] / jnp.maximum(cnts[...][:, 0:1], 1.0)
        out_ref[...] = jnp.dot(
            ro, cw_ref[...], preferred_element_type=jnp.float32) + cb_ref[...]


def _bn_readout_call(h2, mom, g2, be2, batch3d, cw, cb):
    out = cw.shape[1]
    return pl.pallas_call(
        _bn_readout_body,
        grid=(NB,),
        in_specs=[
            pl.BlockSpec((R, D), lambda i: (i, 0)),
            pl.BlockSpec((2, D), lambda i: (0, 0)),
            pl.BlockSpec((1, D), lambda i: (0, 0)),
            pl.BlockSpec((1, D), lambda i: (0, 0)),
            pl.BlockSpec((1, 1, R), lambda i: (i, 0, 0)),
            pl.BlockSpec((D, out), lambda i: (0, 0)),
            pl.BlockSpec((1, out), lambda i: (0, 0)),
        ],
        out_specs=pl.BlockSpec((B, out), lambda i: (0, 0)),
        out_shape=jax.ShapeDtypeStruct((B, out), jnp.float32),
        scratch_shapes=[
            pltpu.VMEM((B, D), jnp.float32),
            pltpu.VMEM((B, 128), jnp.float32),
        ],
    )(h2, mom, g2.reshape(1, D), be2.reshape(1, D), batch3d, cw,
      cb.reshape(1, out))


def kernel(x, edge_index, batch, params):
    e = edge_index.shape[1]
    ept = -(-e // (NW * 2 * CH)) * 2 * CH
    e_pad = ept * NW
    npad = N_AGG - N

    src = edge_index[0]
    dst = edge_index[1]
    pad_idx = jnp.arange(e_pad - e, dtype=jnp.int32) % npad
    src2d = jnp.concatenate([src, pad_idx]).reshape(NW * 2, ept // 2)
    dst3d = jnp.concatenate([dst, pad_idx + N]).reshape(
        NW * 2, ept // (2 * CH), CH)

    batch3d = batch.astype(jnp.int32).reshape(NB, 1, R)

    h = x
    layers = params['layers']
    for li, p in enumerate(layers):
        agg = _sc_scatter_call(h, src2d, dst3d)
        z1, mom1 = _mlp1_call(h, agg, p['W1'], p['b1'])
        h2, mom2 = _mlp2_call(z1, mom1, p['g1'], p['be1'], p['W2'], p['b2'])
        if li != len(layers) - 1:
            h = _bn_call(h2, mom2, p['g2'], p['be2'])
        else:
            return _bn_readout_call(h2, mom2, p['g2'], p['be2'], batch3d,
                                    params['cls_W'], params['cls_b'])
